# baseline (speedup 1.0000x reference)
import functools

import jax
import jax.numpy as jnp
from jax import lax
from jax.experimental import pallas as pl
from jax.experimental.pallas import tpu as pltpu
from jax.experimental.pallas import tpu_sc as plsc

N_POINTS = 320000
D_IN = 128
HID = 64
LAT = 64
B_SEG = 10000

NC = 2
NS = 16
NW = NC * NS
SEG_PER_W = 320
SEG_PAD = NW * SEG_PER_W
CHUNK = 128
CHUNK_SHIFT = 7
SLABS = 2
SEG_SLAB = SEG_PER_W // SLABS
ACC_ROWS = SEG_SLAB
FMIN = float(jnp.finfo(jnp.float32).min)

NHALF = N_POINTS // 2
PHI_BLK = 3200


def _phi_body(x_ref, w1_ref, b1_ref, w2_ref, b2_ref, out_ref):
    h = jnp.dot(x_ref[...], w1_ref[...], preferred_element_type=jnp.float32)
    h = jnp.maximum(h + b1_ref[...], 0.0)
    out_ref[...] = (
        jnp.dot(h, w2_ref[...], preferred_element_type=jnp.float32)
        + b2_ref[...]
    )


def _phi_half(x, w1, b1, w2, b2, half):
    off = half * (NHALF // PHI_BLK)
    return pl.pallas_call(
        _phi_body,
        grid=(NHALF // PHI_BLK,),
        in_specs=[
            pl.BlockSpec((PHI_BLK, D_IN), lambda i: (i + off, 0)),
            pl.BlockSpec((D_IN, HID), lambda i: (0, 0)),
            pl.BlockSpec((1, HID), lambda i: (0, 0)),
            pl.BlockSpec((HID, LAT), lambda i: (0, 0)),
            pl.BlockSpec((1, LAT), lambda i: (0, 0)),
        ],
        out_specs=pl.BlockSpec((PHI_BLK, LAT), lambda i: (i, 0)),
        out_shape=jax.ShapeDtypeStruct((NHALF, LAT), jnp.float32),
    )(x, w1, b1.reshape(1, HID), w2, b2.reshape(1, LAT))


def _make_seg_body(p0, p1):
    nblk16 = (p1 - p0) // 16

    def _seg_body(enc_hbm, ids_hbm, sum_hbm, max_hbm, cnt_hbm,
                  chunk_v, ids_v, chunk_w, ids_w, sum_acc, max_acc, cnt_acc,
                  probe_v, sem0, sem1):
        wid = lax.axis_index("s") * NC + lax.axis_index("c")
        seg_lo = wid * SEG_PER_W

        def count_less(v, t):
            cnt = jnp.int32(0)
            for l in range(16):
                cnt = cnt + jnp.where(v[l] < t, jnp.int32(1), jnp.int32(0))
            return cnt

        def lower_bound(t):
            lo = jnp.int32(p0 // 16)
            rng = jnp.int32(nblk16)
            lanes = lax.iota(jnp.int32, 16)
            for _ in range(4):
                step = (rng + 15) >> 4
                pos = jnp.minimum(lo + lanes * step, lo + rng - 1)
                pltpu.async_copy(ids_hbm.at[pos * 16], probe_v, sem0).wait()
                cnt = jnp.maximum(count_less(probe_v[...], t), 1)
                lo = lo + (cnt - 1) * step
                rng = jnp.minimum(step, rng - (cnt - 1) * step)
            pltpu.sync_copy(
                ids_hbm.at[pl.ds(pl.multiple_of(lo * 16, 16), 16)], probe_v)
            return lo * 16 + count_less(probe_v[...], t)

        zero16 = jnp.zeros((16,), jnp.float32)
        neg16 = jnp.full((16,), FMIN, jnp.float32)

        def do_slab(t0, p_lo):
            t1 = jnp.maximum(jnp.minimum(t0 + SEG_SLAB, B_SEG), t0)
            p_hi = lower_bound(t1)

            def initrow(i, _):
                for j in range(LAT // 16):
                    sl = pl.ds(i * LAT + j * 16, 16)
                    sum_acc[sl] = zero16
                    max_acc[sl] = neg16
                cnt_acc[pl.ds(i * 16, 16)] = zero16
                return 0
            lax.fori_loop(0, ACC_ROWS, initrow, 0)

            c0 = p_lo >> CHUNK_SHIFT
            c1 = (p_hi + CHUNK - 1) >> CHUNK_SHIFT

            def flush(cs, rc, svecs, mvecs):
                @pl.when((cs >= t0) & (cs < t1))
                def _():
                    ls = cs - t0
                    for j in range(LAT // 16):
                        sl = pl.ds(ls * LAT + j * 16, 16)
                        sum_acc[sl] = svecs[j]
                        max_acc[sl] = mvecs[j]
                    cnt_acc[pl.ds(ls * 16, 16)] = (
                        jnp.broadcast_to(rc, (16,)).astype(jnp.float32))

            def process(c_ref, i_ref, carry):
                def grp_body(gi, carry):
                    cs, rc, s0, s1, s2, s3, m0, m1, m2, m3 = carry
                    svecs, mvecs = [s0, s1, s2, s3], [m0, m1, m2, m3]
                    r0 = gi * 16
                    idvec = i_ref[pl.ds(r0, 16)]
                    for l in range(16):
                        sid = idvec[l]
                        changed = sid != cs
                        rows = [c_ref[r0 + l, pl.ds(j * 16, 16)]
                                for j in range(LAT // 16)]
                        @pl.when(changed)
                        def _(cs=cs, rc=rc, svecs=svecs, mvecs=mvecs):
                            flush(cs, rc, svecs, mvecs)
                        kv = jnp.broadcast_to(
                            (sid == cs).astype(jnp.int32), (16,)
                        ).astype(jnp.float32)
                        kfmin = (1.0 - kv) * FMIN
                        svecs = [rows[j] + kv * svecs[j]
                                 for j in range(LAT // 16)]
                        mvecs = [jnp.maximum(rows[j], kv * mvecs[j] + kfmin)
                                 for j in range(LAT // 16)]
                        rc = jnp.where(changed, jnp.int32(1), rc + 1)
                        cs = sid
                    return (cs, rc, *svecs, *mvecs)
                return lax.fori_loop(0, CHUNK // 16, grp_body, carry)

            def ebase(ci):
                return pl.multiple_of((ci << CHUNK_SHIFT) - p0, CHUNK)

            def ibase(ci):
                return pl.multiple_of(ci << CHUNK_SHIFT, CHUNK)

            def start(ci, c_ref, i_ref, sem):
                pltpu.async_copy(
                    enc_hbm.at[pl.ds(ebase(ci), CHUNK)], c_ref, sem)
                pltpu.async_copy(
                    ids_hbm.at[pl.ds(ibase(ci), CHUNK)], i_ref, sem)

            def wait(ci, c_ref, i_ref, sem):
                pltpu.make_async_copy(
                    enc_hbm.at[pl.ds(ebase(ci), CHUNK)], c_ref, sem).wait()
                pltpu.make_async_copy(
                    ids_hbm.at[pl.ds(ibase(ci), CHUNK)], i_ref, sem).wait()

            odd = (c1 - c0) & 1
            can_left = c0 > (p0 // CHUNK)
            c0e = jnp.where((odd == 1) & can_left, c0 - 1, c0)
            c1e = jnp.where((odd == 1) & (~can_left), c1 + 1, c1)
            npairs = (c1e - c0e) >> 1

            @pl.when(npairs > 0)
            def _():
                start(c0e, chunk_v, ids_v, sem0)

            def pair_body(k, carry):
                a = c0e + 2 * k
                wait(a, chunk_v, ids_v, sem0)
                start(a + 1, chunk_w, ids_w, sem1)
                carry = process(chunk_v, ids_v, carry)
                wait(a + 1, chunk_w, ids_w, sem1)
                @pl.when(a + 2 < c1e)
                def _():
                    start(a + 2, chunk_v, ids_v, sem0)
                carry = process(chunk_w, ids_w, carry)
                return carry

            carry0 = (jnp.int32(-1), jnp.int32(0)) + (zero16,) * 8
            carry = lax.fori_loop(0, npairs, pair_body, carry0)
            cs, rc = carry[0], carry[1]
            flush(cs, rc, carry[2:6], carry[6:10])

            out_lo = pl.multiple_of(t0, 16)
            pltpu.sync_copy(
                sum_acc.at[pl.ds(0, SEG_SLAB * LAT)],
                sum_hbm.at[pl.ds(pl.multiple_of(out_lo * LAT, 1024),
                                 SEG_SLAB * LAT)])
            pltpu.sync_copy(
                max_acc.at[pl.ds(0, SEG_SLAB * LAT)],
                max_hbm.at[pl.ds(pl.multiple_of(out_lo * LAT, 1024),
                                 SEG_SLAB * LAT)])
            pltpu.sync_copy(
                cnt_acc.at[pl.ds(0, SEG_SLAB * 16)],
                cnt_hbm.at[pl.ds(pl.multiple_of(out_lo * 16, 256),
                                 SEG_SLAB * 16)])
            return p_hi

        p = lower_bound(seg_lo)
        for s in range(SLABS):
            p = do_slab(seg_lo + s * SEG_SLAB, p)

    return _seg_body


def _make_seg_pool(p0, p1):
    return functools.partial(
        pl.kernel,
        out_type=(
            jax.ShapeDtypeStruct((SEG_PAD * LAT,), jnp.float32),
            jax.ShapeDtypeStruct((SEG_PAD * LAT,), jnp.float32),
            jax.ShapeDtypeStruct((SEG_PAD * 16,), jnp.float32),
        ),
        mesh=plsc.VectorSubcoreMesh(
            core_axis_name="c", subcore_axis_name="s",
            num_cores=NC, num_subcores=NS),
        scratch_types=[
            pltpu.VMEM((CHUNK, LAT), jnp.float32),
            pltpu.VMEM((CHUNK,), jnp.int32),
            pltpu.VMEM((CHUNK, LAT), jnp.float32),
            pltpu.VMEM((CHUNK,), jnp.int32),
            pltpu.VMEM((ACC_ROWS * LAT,), jnp.float32),
            pltpu.VMEM((ACC_ROWS * LAT,), jnp.float32),
            pltpu.VMEM((ACC_ROWS * 16,), jnp.float32),
            pltpu.VMEM((16,), jnp.int32),
            pltpu.SemaphoreType.DMA,
            pltpu.SemaphoreType.DMA,
        ],
    )(_make_seg_body(p0, p1))


_seg_pool_a = _make_seg_pool(0, NHALF)
_seg_pool_b = _make_seg_pool(NHALF, N_POINTS)


def _rho_body(sa_ref, sb_ref, ma_ref, mb_ref, ca_ref, cb_ref,
              wa_ref, wb_ref, wc_ref, b1_ref, w2_ref, b2_ref, out_ref):
    psum = sa_ref[...][:B_SEG] + sb_ref[...][:B_SEG]
    pmax = jnp.maximum(ma_ref[...][:B_SEG], mb_ref[...][:B_SEG])
    cnt = jnp.maximum(ca_ref[...][:B_SEG, 0:1] + cb_ref[...][:B_SEG, 0:1],
                      1.0)
    mean = psum / cnt
    g = (jnp.dot(mean, wa_ref[...], preferred_element_type=jnp.float32)
         + jnp.dot(pmax, wb_ref[...], preferred_element_type=jnp.float32)
         + jnp.log(cnt) * wc_ref[...]
         + b1_ref[...])
    out_ref[...] = (
        jnp.dot(jnp.maximum(g, 0.0), w2_ref[...],
                preferred_element_type=jnp.float32)
        + b2_ref[...]
    )


def _rho(sa, sb, ma, mb, ca, cb, rho_W1, rho_b1, rho_W2, rho_b2):
    full = lambda s: pl.BlockSpec(s, lambda: tuple(0 for _ in s))
    return pl.pallas_call(
        _rho_body,
        in_specs=[
            full((SEG_PAD, LAT)), full((SEG_PAD, LAT)),
            full((SEG_PAD, LAT)), full((SEG_PAD, LAT)),
            full((SEG_PAD, 16)), full((SEG_PAD, 16)),
            full((LAT, HID)), full((LAT, HID)), full((1, HID)),
            full((1, HID)), full((HID, 1)), full((1, 1)),
        ],
        out_specs=full((B_SEG, 1)),
        out_shape=jax.ShapeDtypeStruct((B_SEG, 1), jnp.float32),
    )(sa, sb, ma, mb, ca, cb,
      rho_W1[:LAT], rho_W1[LAT:2 * LAT], rho_W1[2 * LAT:],
      rho_b1.reshape(1, HID), rho_W2, rho_b2.reshape(1, 1))


def kernel(x, batch_index, phi_W1, phi_b1, phi_W2, phi_b2,
           rho_W1, rho_b1, rho_W2, rho_b2):
    ids = batch_index.astype(jnp.int32)
    enc_a = _phi_half(x, phi_W1, phi_b1, phi_W2, phi_b2, 0)
    sa, ma, ca = _seg_pool_a(enc_a, ids)
    enc_b = _phi_half(x, phi_W1, phi_b1, phi_W2, phi_b2, 1)
    sb, mb, cb = _seg_pool_b(enc_b, ids)
    out = _rho(sa.reshape(SEG_PAD, LAT), sb.reshape(SEG_PAD, LAT),
               ma.reshape(SEG_PAD, LAT), mb.reshape(SEG_PAD, LAT),
               ca.reshape(SEG_PAD, 16), cb.reshape(SEG_PAD, 16),
               rho_W1, rho_b1, rho_W2, rho_b2)
    return out.reshape(-1)


# speedup vs baseline: 1.3530x; 1.3530x over previous
import functools

import jax
import jax.numpy as jnp
from jax import lax
from jax.experimental import pallas as pl
from jax.experimental.pallas import tpu as pltpu
from jax.experimental.pallas import tpu_sc as plsc

N_POINTS = 320000
D_IN = 128
HID = 64
LAT = 64
B_SEG = 10000

NC = 2
NS = 16
NW = NC * NS
SEG_PER_W = 320
SEG_PAD = NW * SEG_PER_W
CHUNK = 256
CHUNK_SHIFT = 8
SLABS = 2
SEG_SLAB = SEG_PER_W // SLABS
ACC_ROWS = SEG_SLAB
FMIN = float(jnp.finfo(jnp.float32).min)

NHALF = N_POINTS // 2
PHI_BLK = 3200


def _phi_body(x_ref, w1_ref, b1_ref, w2_ref, b2_ref, out_ref):
    h = jnp.dot(x_ref[...], w1_ref[...], preferred_element_type=jnp.float32)
    h = jnp.maximum(h + b1_ref[...], 0.0)
    out_ref[...] = (
        jnp.dot(h, w2_ref[...], preferred_element_type=jnp.float32)
        + b2_ref[...]
    )


def _phi(x, w1, b1, w2, b2):
    return pl.pallas_call(
        _phi_body,
        grid=(N_POINTS // PHI_BLK,),
        in_specs=[
            pl.BlockSpec((PHI_BLK, D_IN), lambda i: (i, 0)),
            pl.BlockSpec((D_IN, HID), lambda i: (0, 0)),
            pl.BlockSpec((1, HID), lambda i: (0, 0)),
            pl.BlockSpec((HID, LAT), lambda i: (0, 0)),
            pl.BlockSpec((1, LAT), lambda i: (0, 0)),
        ],
        out_specs=pl.BlockSpec((PHI_BLK, LAT), lambda i: (i, 0)),
        out_shape=jax.ShapeDtypeStruct((N_POINTS, LAT), jnp.float32),
    )(x, w1, b1.reshape(1, HID), w2, b2.reshape(1, LAT))


def _make_seg_body(p0, p1):
    nblk16 = (p1 - p0) // 16

    def _seg_body(enc_hbm, ids_hbm, sum_hbm, max_hbm, cnt_hbm,
                  chunk_v, ids_v, chunk_w, ids_w, sum_acc, max_acc, cnt_acc,
                  probe_v, sem0, sem1):
        wid = lax.axis_index("s") * NC + lax.axis_index("c")
        seg_lo = wid * SEG_PER_W

        def count_less(v, t):
            cnt = jnp.int32(0)
            for l in range(16):
                cnt = cnt + jnp.where(v[l] < t, jnp.int32(1), jnp.int32(0))
            return cnt

        def lower_bound(t):
            lo = jnp.int32(p0 // 16)
            rng = jnp.int32(nblk16)
            lanes = lax.iota(jnp.int32, 16)
            for _ in range(4):
                step = (rng + 15) >> 4
                pos = jnp.minimum(lo + lanes * step, lo + rng - 1)
                pltpu.async_copy(ids_hbm.at[pos * 16], probe_v, sem0).wait()
                cnt = jnp.maximum(count_less(probe_v[...], t), 1)
                lo = lo + (cnt - 1) * step
                rng = jnp.minimum(step, rng - (cnt - 1) * step)
            pltpu.sync_copy(
                ids_hbm.at[pl.ds(pl.multiple_of(lo * 16, 16), 16)], probe_v)
            return lo * 16 + count_less(probe_v[...], t)

        zero16 = jnp.zeros((16,), jnp.float32)
        neg16 = jnp.full((16,), FMIN, jnp.float32)

        def do_slab(t0, p_lo):
            t1 = jnp.maximum(jnp.minimum(t0 + SEG_SLAB, B_SEG), t0)
            p_hi = lower_bound(t1)

            def initrow(i, _):
                for j in range(LAT // 16):
                    sl = pl.ds(i * LAT + j * 16, 16)
                    sum_acc[sl] = zero16
                    max_acc[sl] = neg16
                cnt_acc[pl.ds(i * 16, 16)] = zero16
                return 0
            lax.fori_loop(0, ACC_ROWS, initrow, 0)

            c0 = p_lo >> CHUNK_SHIFT
            c1 = (p_hi + CHUNK - 1) >> CHUNK_SHIFT

            def flush(cs, rc, svecs, mvecs):
                @pl.when((cs >= t0) & (cs < t1))
                def _():
                    ls = cs - t0
                    for j in range(LAT // 16):
                        sl = pl.ds(ls * LAT + j * 16, 16)
                        sum_acc[sl] = svecs[j]
                        max_acc[sl] = mvecs[j]
                    cnt_acc[pl.ds(ls * 16, 16)] = (
                        jnp.broadcast_to(rc, (16,)).astype(jnp.float32))

            def process(c_ref, i_ref, carry):
                def grp_body(gi, carry):
                    cs, rc, s0, s1, s2, s3, m0, m1, m2, m3 = carry
                    svecs, mvecs = [s0, s1, s2, s3], [m0, m1, m2, m3]
                    r0 = gi * 16
                    idvec = i_ref[pl.ds(r0, 16)]
                    for l in range(16):
                        sid = idvec[l]
                        changed = sid != cs
                        rows = [c_ref[r0 + l, pl.ds(j * 16, 16)]
                                for j in range(LAT // 16)]
                        @pl.when(changed)
                        def _(cs=cs, rc=rc, svecs=svecs, mvecs=mvecs):
                            flush(cs, rc, svecs, mvecs)
                        kv = jnp.broadcast_to(
                            (sid == cs).astype(jnp.int32), (16,)
                        ).astype(jnp.float32)
                        kfmin = (1.0 - kv) * FMIN
                        svecs = [rows[j] + kv * svecs[j]
                                 for j in range(LAT // 16)]
                        mvecs = [jnp.maximum(rows[j], kv * mvecs[j] + kfmin)
                                 for j in range(LAT // 16)]
                        rc = jnp.where(changed, jnp.int32(1), rc + 1)
                        cs = sid
                    return (cs, rc, *svecs, *mvecs)
                return lax.fori_loop(0, CHUNK // 16, grp_body, carry)

            def ebase(ci):
                return pl.multiple_of((ci << CHUNK_SHIFT) - p0, CHUNK)

            def ibase(ci):
                return pl.multiple_of(ci << CHUNK_SHIFT, CHUNK)

            def start(ci, c_ref, i_ref, sem):
                pltpu.async_copy(
                    enc_hbm.at[pl.ds(ebase(ci), CHUNK)], c_ref, sem)
                pltpu.async_copy(
                    ids_hbm.at[pl.ds(ibase(ci), CHUNK)], i_ref, sem)

            def wait(ci, c_ref, i_ref, sem):
                pltpu.make_async_copy(
                    enc_hbm.at[pl.ds(ebase(ci), CHUNK)], c_ref, sem).wait()
                pltpu.make_async_copy(
                    ids_hbm.at[pl.ds(ibase(ci), CHUNK)], i_ref, sem).wait()

            odd = (c1 - c0) & 1
            can_left = c0 > (p0 // CHUNK)
            c0e = jnp.where((odd == 1) & can_left, c0 - 1, c0)
            c1e = jnp.where((odd == 1) & (~can_left), c1 + 1, c1)
            npairs = (c1e - c0e) >> 1

            @pl.when(npairs > 0)
            def _():
                start(c0e, chunk_v, ids_v, sem0)

            def pair_body(k, carry):
                a = c0e + 2 * k
                wait(a, chunk_v, ids_v, sem0)
                start(a + 1, chunk_w, ids_w, sem1)
                carry = process(chunk_v, ids_v, carry)
                wait(a + 1, chunk_w, ids_w, sem1)
                @pl.when(a + 2 < c1e)
                def _():
                    start(a + 2, chunk_v, ids_v, sem0)
                carry = process(chunk_w, ids_w, carry)
                return carry

            carry0 = (jnp.int32(-1), jnp.int32(0)) + (zero16,) * 8
            carry = lax.fori_loop(0, npairs, pair_body, carry0)
            cs, rc = carry[0], carry[1]
            flush(cs, rc, carry[2:6], carry[6:10])

            out_lo = pl.multiple_of(t0, 16)
            pltpu.sync_copy(
                sum_acc.at[pl.ds(0, SEG_SLAB * LAT)],
                sum_hbm.at[pl.ds(pl.multiple_of(out_lo * LAT, 1024),
                                 SEG_SLAB * LAT)])
            pltpu.sync_copy(
                max_acc.at[pl.ds(0, SEG_SLAB * LAT)],
                max_hbm.at[pl.ds(pl.multiple_of(out_lo * LAT, 1024),
                                 SEG_SLAB * LAT)])
            pltpu.sync_copy(
                cnt_acc.at[pl.ds(0, SEG_SLAB * 16)],
                cnt_hbm.at[pl.ds(pl.multiple_of(out_lo * 16, 256),
                                 SEG_SLAB * 16)])
            return p_hi

        p = lower_bound(seg_lo)
        for s in range(SLABS):
            p = do_slab(seg_lo + s * SEG_SLAB, p)

    return _seg_body


def _make_seg_pool(p0, p1):
    return functools.partial(
        pl.kernel,
        out_type=(
            jax.ShapeDtypeStruct((SEG_PAD * LAT,), jnp.float32),
            jax.ShapeDtypeStruct((SEG_PAD * LAT,), jnp.float32),
            jax.ShapeDtypeStruct((SEG_PAD * 16,), jnp.float32),
        ),
        mesh=plsc.VectorSubcoreMesh(
            core_axis_name="c", subcore_axis_name="s",
            num_cores=NC, num_subcores=NS),
        scratch_types=[
            pltpu.VMEM((CHUNK, LAT), jnp.float32),
            pltpu.VMEM((CHUNK,), jnp.int32),
            pltpu.VMEM((CHUNK, LAT), jnp.float32),
            pltpu.VMEM((CHUNK,), jnp.int32),
            pltpu.VMEM((ACC_ROWS * LAT,), jnp.float32),
            pltpu.VMEM((ACC_ROWS * LAT,), jnp.float32),
            pltpu.VMEM((ACC_ROWS * 16,), jnp.float32),
            pltpu.VMEM((16,), jnp.int32),
            pltpu.SemaphoreType.DMA,
            pltpu.SemaphoreType.DMA,
        ],
    )(_make_seg_body(p0, p1))


_seg_pool = _make_seg_pool(0, N_POINTS)


def _rho_body(sa_ref, ma_ref, ca_ref,
              wa_ref, wb_ref, wc_ref, b1_ref, w2_ref, b2_ref, out_ref):
    psum = sa_ref[...][:B_SEG]
    pmax = ma_ref[...][:B_SEG]
    cnt = jnp.maximum(ca_ref[...][:B_SEG, 0:1], 1.0)
    mean = psum / cnt
    g = (jnp.dot(mean, wa_ref[...], preferred_element_type=jnp.float32)
         + jnp.dot(pmax, wb_ref[...], preferred_element_type=jnp.float32)
         + jnp.log(cnt) * wc_ref[...]
         + b1_ref[...])
    out_ref[...] = (
        jnp.dot(jnp.maximum(g, 0.0), w2_ref[...],
                preferred_element_type=jnp.float32)
        + b2_ref[...]
    )


def _rho(sa, ma, ca, rho_W1, rho_b1, rho_W2, rho_b2):
    full = lambda s: pl.BlockSpec(s, lambda: tuple(0 for _ in s))
    return pl.pallas_call(
        _rho_body,
        in_specs=[
            full((SEG_PAD, LAT)), full((SEG_PAD, LAT)), full((SEG_PAD, 16)),
            full((LAT, HID)), full((LAT, HID)), full((1, HID)),
            full((1, HID)), full((HID, 1)), full((1, 1)),
        ],
        out_specs=full((B_SEG, 1)),
        out_shape=jax.ShapeDtypeStruct((B_SEG, 1), jnp.float32),
    )(sa, ma, ca,
      rho_W1[:LAT], rho_W1[LAT:2 * LAT], rho_W1[2 * LAT:],
      rho_b1.reshape(1, HID), rho_W2, rho_b2.reshape(1, 1))


def kernel(x, batch_index, phi_W1, phi_b1, phi_W2, phi_b2,
           rho_W1, rho_b1, rho_W2, rho_b2):
    ids = batch_index.astype(jnp.int32)
    enc = _phi(x, phi_W1, phi_b1, phi_W2, phi_b2)
    sa, ma, ca = _seg_pool(enc, ids)
    out = _rho(sa.reshape(SEG_PAD, LAT), ma.reshape(SEG_PAD, LAT),
               ca.reshape(SEG_PAD, 16),
               rho_W1, rho_b1, rho_W2, rho_b2)
    return out.reshape(-1)
